# baseline (speedup 1.0000x reference)
import functools

import jax
import jax.numpy as jnp
from jax import lax
from jax.experimental import pallas as pl
from jax.experimental.pallas import tpu as pltpu
from jax.experimental.pallas import tpu_sc as plsc

NSEG = 512
D = 128
NC = 2
NS = 16
NW = NC * NS
SEG_PER_W = NSEG // NW
NPOW = 5
B = 256
NBUF = 2
SEG_STRIDE = NPOW * D
OFFS_PAD = 544
CH1 = 10000
CH2 = 5008
ETAB = 544


def _sc_run_ends(ids1p, ids2p):
    mesh = plsc.VectorSubcoreMesh(core_axis_name="c", subcore_axis_name="s")

    @functools.partial(
        pl.kernel,
        out_type=jax.ShapeDtypeStruct((2 * NW * ETAB,), jnp.int32),
        mesh=mesh,
        compiler_params=pltpu.CompilerParams(needs_layout_passes=False),
        scratch_types=[
            pltpu.VMEM((CH1 + 16,), jnp.int32),
            pltpu.VMEM((ETAB,), jnp.int32),
        ],
    )
    def k(i1_hbm, i2_hbm, e_hbm, idsv, ends):
        wid = lax.axis_index("s") * NC + lax.axis_index("c")
        iota = lax.iota(jnp.int32, 16)
        zero = jnp.zeros((16,), jnp.int32)
        for t, (src, ch) in enumerate(((i1_hbm, CH1), (i2_hbm, CH2))):
            pltpu.sync_copy(src.at[pl.ds(wid * ch, ch + 16)],
                            idsv.at[pl.ds(0, ch + 16)])
            for i in range(ETAB // 16):
                ends[pl.ds(16 * i, 16)] = zero
            base = wid * ch + 1

            def grp(i, carry, base=base):
                g = idsv[pl.ds(16 * i, 16)]
                gn = idsv[pl.ds(16 * i + 1, 16)]
                pos = iota + (16 * i + base)
                plsc.store_scatter(ends, [g], pos, mask=g != gn)
                return carry

            lax.fori_loop(0, ch // 16, grp, 0)
            pltpu.sync_copy(
                ends, e_hbm.at[pl.ds((t * NW + wid) * ETAB, ETAB)])

    return k(ids1p, ids2p)


def _offsets(ids, chunk):
    n = ids.shape[0]
    pad = NW * chunk + 16 - n
    return jnp.concatenate([ids, jnp.full((pad,), NSEG, jnp.int32)])


def _accumulate_task(x_hbm, offs_v, out_hbm, bufs, sems,
                     stage, offs_s, n_rows, wid):
    g0 = offs_v[pl.ds(wid * 16, 16)]
    g1 = offs_v[pl.ds(wid * 16 + 8, 16)]
    for k in range(SEG_PER_W):
        offs_s[k] = g0[k]
    offs_s[SEG_PER_W] = g1[8]
    rs = g0[0]
    re = g1[8]
    nbw = (re - rs + B - 1) // B

    zf = jnp.zeros((16,), jnp.float32)

    def zero_body(i, c):
        stage[pl.ds(16 * i, 16)] = zf
        return c

    lax.fori_loop(0, SEG_PER_W * SEG_STRIDE // 16, zero_body, 0)

    def start(blk, b):
        w0 = rs + blk * B
        c0 = jnp.minimum(w0, n_rows - B)
        pltpu.async_copy(x_hbm.at[pl.ds(c0 * D, B * D)], bufs[b], sems[b])

    def wait(b):
        pltpu.make_async_copy(x_hbm.at[pl.ds(0, B * D)], bufs[b], sems[b]).wait()

    for b0 in range(NBUF):
        @pl.when(nbw > b0)
        def _(b0=b0):
            start(b0, b0)

    def pair_body(i, carry):
        for b in range(NBUF):
            blk = NBUF * i + b

            @pl.when(blk < nbw)
            def _(blk=blk, b=b):
                buf = bufs[b]
                wait(b)
                w0 = rs + blk * B
                c0 = jnp.minimum(w0, n_rows - B)
                hi_w = w0 + B

                def seg_body(k, c):
                    a = jnp.maximum(offs_s[k], w0)
                    bb = jnp.minimum(offs_s[k + 1], hi_w)
                    lo = a - c0
                    hi = bb - c0
                    sbase = k * SEG_STRIDE

                    def row_body(r, acc):
                        base = r * D
                        na = list(acc)
                        for g in range(8):
                            v = buf[pl.ds(base + 16 * g, 16)]
                            plsc.addupdate(stage.at[pl.ds(sbase + 16 * g, 16)], v)
                            v2 = v * v
                            plsc.addupdate(
                                stage.at[pl.ds(sbase + D + 16 * g, 16)], v2)
                            v4 = v2 * v2
                            na[0 * 8 + g] = na[0 * 8 + g] + v2 * v
                            na[1 * 8 + g] = na[1 * 8 + g] + v4
                            na[2 * 8 + g] = na[2 * 8 + g] + v4 * v
                        return tuple(na)

                    acc = lax.fori_loop(lo, hi, row_body, (zf,) * 24)

                    @pl.when(hi > lo)
                    def _():
                        for j in range(3):
                            for g in range(8):
                                idx = sbase + (j + 2) * D + 16 * g
                                plsc.addupdate(
                                    stage.at[pl.ds(idx, 16)], acc[j * 8 + g])

                    return c

                lax.fori_loop(0, SEG_PER_W, seg_body, 0)

                @pl.when(blk + NBUF < nbw)
                def _():
                    start(blk + NBUF, b)

        return carry

    lax.fori_loop(0, (nbw + NBUF - 1) // NBUF, pair_body, 0)

    pltpu.sync_copy(
        stage, out_hbm.at[pl.ds(wid * SEG_PER_W * SEG_STRIDE, SEG_PER_W * SEG_STRIDE)]
    )


def _sc_moment_sums(x1f, x2f, offsp):
    n1 = x1f.shape[0] // D
    n2 = x2f.shape[0] // D
    mesh = plsc.VectorSubcoreMesh(core_axis_name="c", subcore_axis_name="s")

    @functools.partial(
        pl.kernel,
        out_type=(
            jax.ShapeDtypeStruct((NSEG * SEG_STRIDE,), jnp.float32),
            jax.ShapeDtypeStruct((NSEG * SEG_STRIDE,), jnp.float32),
        ),
        mesh=mesh,
        scratch_types=(
            [pltpu.VMEM((B * D,), jnp.float32)] * NBUF
            + [
                pltpu.VMEM((SEG_PER_W * SEG_STRIDE,), jnp.float32),
                pltpu.VMEM((2 * OFFS_PAD,), jnp.int32),
                pltpu.SMEM((32,), jnp.int32),
            ]
            + [pltpu.SemaphoreType.DMA] * NBUF
        ),
    )
    def sc_kernel(x1_hbm, x2_hbm, o_hbm, s1_hbm, s2_hbm, *rest):
        bufs = rest[:NBUF]
        stage, o_v, offs_s = rest[NBUF:NBUF + 3]
        sems = rest[NBUF + 3:]
        wid = lax.axis_index("s") * NC + lax.axis_index("c")
        pltpu.sync_copy(o_hbm, o_v)
        _accumulate_task(x1_hbm, o_v.at[pl.ds(0, OFFS_PAD)], s1_hbm,
                         bufs, sems, stage, offs_s, n1, wid)
        _accumulate_task(x2_hbm, o_v.at[pl.ds(OFFS_PAD, OFFS_PAD)], s2_hbm,
                         bufs, sems, stage, offs_s, n2, wid)

    return sc_kernel(x1f, x2f, offsp)


def _tc_finish(s1, s2, lo1, hi1, lo2, hi2):
    def body(s1_ref, s2_ref, lo1_ref, hi1_ref, lo2_ref, hi2_ref, out_ref):
        def central(s_ref, lo_ref, hi_ref):
            n = jnp.maximum((hi_ref[...] - lo_ref[...]).astype(jnp.float32), 1.0)
            inv = 1.0 / n
            M1 = s_ref[:, 0 * D:1 * D] * inv
            M2 = s_ref[:, 1 * D:2 * D] * inv
            M3 = s_ref[:, 2 * D:3 * D] * inv
            M4 = s_ref[:, 3 * D:4 * D] * inv
            M5 = s_ref[:, 4 * D:5 * D] * inv
            m = M1
            m2 = m * m
            m3 = m2 * m
            c2 = M2 - m2
            c3 = M3 - 3.0 * m * M2 + 2.0 * m3
            c4 = M4 - 4.0 * m * M3 + 6.0 * m2 * M2 - 3.0 * m2 * m2
            c5 = M5 - 5.0 * m * M4 + 10.0 * m2 * M3 - 10.0 * m3 * M2 + 4.0 * m3 * m2
            return (m, c2, c3, c4, c5)

        A = central(s1_ref, lo1_ref, hi1_ref)
        Bm = central(s2_ref, lo2_ref, hi2_ref)
        tot = jnp.zeros((NSEG, 1), jnp.float32)
        for a, b in zip(A, Bm):
            diff = a - b
            tot = tot + jnp.sqrt(jnp.sum(diff * diff, axis=1, keepdims=True))
        out_ref[...] = (jnp.sum(tot) / NSEG) * jnp.ones((1, 1), jnp.float32)

    return pl.pallas_call(
        body,
        out_shape=jax.ShapeDtypeStruct((1, 1), jnp.float32),
    )(s1, s2, lo1, hi1, lo2, hi2)


def kernel(x1, x2, og_batch, coarse_batch, n_moments):
    ids1 = og_batch.astype(jnp.int32)
    ids2 = coarse_batch.astype(jnp.int32)

    e = _sc_run_ends(_offsets(ids1, CH1), _offsets(ids2, CH2))
    ends = jnp.max(e.reshape(2, NW, ETAB), axis=1)[:, :512]
    offs = jnp.concatenate(
        [jnp.zeros((2, 1), jnp.int32), lax.cummax(ends, axis=1)], axis=1)
    offsp = jnp.zeros((2, OFFS_PAD), jnp.int32).at[:, :513].set(offs)

    s1, s2 = _sc_moment_sums(x1.reshape(-1), x2.reshape(-1),
                             offsp.reshape(-1))

    out = _tc_finish(
        s1.reshape(NSEG, SEG_STRIDE),
        s2.reshape(NSEG, SEG_STRIDE),
        offs[0, :512].reshape(NSEG, 1),
        offs[0, 1:].reshape(NSEG, 1),
        offs[1, :512].reshape(NSEG, 1),
        offs[1, 1:].reshape(NSEG, 1),
    )
    return out[0, 0]


# speedup vs baseline: 2.2606x; 2.2606x over previous
import functools

import jax
import jax.numpy as jnp
from jax import lax
from jax.experimental import pallas as pl
from jax.experimental.pallas import tpu as pltpu
from jax.experimental.pallas import tpu_sc as plsc

NSEG = 512
D = 128
NC = 2
NS = 16
NW = NC * NS
SEG_PER_W = NSEG // NW
NPOW = 5
B = 256
NBUF = 2
SEG_STRIDE = NPOW * D
OFFS_PAD = 544
CH1 = 10000
CH2 = 5008
ETAB = 544


def _sc_run_ends(ids1p, ids2p):
    mesh = plsc.VectorSubcoreMesh(core_axis_name="c", subcore_axis_name="s")

    @functools.partial(
        pl.kernel,
        out_type=jax.ShapeDtypeStruct((2 * NW * ETAB,), jnp.int32),
        mesh=mesh,
        compiler_params=pltpu.CompilerParams(needs_layout_passes=False),
        scratch_types=[
            pltpu.VMEM((CH1 + 16,), jnp.int32),
            pltpu.VMEM((ETAB,), jnp.int32),
        ],
    )
    def k(i1_hbm, i2_hbm, e_hbm, idsv, ends):
        wid = lax.axis_index("s") * NC + lax.axis_index("c")
        iota = lax.iota(jnp.int32, 16)
        zero = jnp.zeros((16,), jnp.int32)
        for t, (src, ch) in enumerate(((i1_hbm, CH1), (i2_hbm, CH2))):
            pltpu.sync_copy(src.at[pl.ds(wid * ch, ch + 16)],
                            idsv.at[pl.ds(0, ch + 16)])
            for i in range(ETAB // 16):
                ends[pl.ds(16 * i, 16)] = zero
            base = wid * ch + 1

            def grp(i, carry, base=base):
                g = idsv[pl.ds(16 * i, 16)]
                gn = idsv[pl.ds(16 * i + 1, 16)]
                pos = iota + (16 * i + base)
                plsc.store_scatter(ends, [g], pos, mask=g != gn)
                return carry

            lax.fori_loop(0, ch // 16, grp, 0)
            pltpu.sync_copy(
                ends, e_hbm.at[pl.ds((t * NW + wid) * ETAB, ETAB)])

    return k(ids1p, ids2p)


def _offsets(ids, chunk):
    n = ids.shape[0]
    pad = NW * chunk + 16 - n
    return jnp.concatenate([ids, jnp.full((pad,), NSEG, jnp.int32)])


def _accumulate_task(x_hbm, offs_v, out_hbm, bufs, sems,
                     stage, offs_s, n_rows, wid):
    g0 = offs_v[pl.ds(wid * 16, 16)]
    g1 = offs_v[pl.ds(wid * 16 + 8, 16)]
    for k in range(SEG_PER_W):
        offs_s[k] = g0[k]
    offs_s[SEG_PER_W] = g1[8]
    rs = g0[0]
    re = g1[8]
    nbw = (re - rs + B - 1) // B

    zf = jnp.zeros((16,), jnp.float32)

    def zero_body(i, c):
        stage[pl.ds(16 * i, 16)] = zf
        return c

    lax.fori_loop(0, SEG_PER_W * SEG_STRIDE // 16, zero_body, 0)

    def start(blk, b):
        w0 = rs + blk * B
        c0 = jnp.minimum(w0, n_rows - B)
        pltpu.async_copy(x_hbm.at[pl.ds(c0 * D, B * D)], bufs[b], sems[b])

    def wait(b):
        pltpu.make_async_copy(x_hbm.at[pl.ds(0, B * D)], bufs[b], sems[b]).wait()

    for b0 in range(NBUF):
        @pl.when(nbw > b0)
        def _(b0=b0):
            start(b0, b0)

    def pair_body(i, carry):
        for b in range(NBUF):
            blk = NBUF * i + b

            @pl.when(blk < nbw)
            def _(blk=blk, b=b):
                buf = bufs[b]
                wait(b)
                w0 = rs + blk * B
                c0 = jnp.minimum(w0, n_rows - B)
                hi_w = w0 + B

                def seg_body(k, c):
                    a = jnp.maximum(offs_s[k], w0)
                    bb = jnp.minimum(offs_s[k + 1], hi_w)
                    lo = a - c0
                    hi = bb - c0

                    def row_body(r, acc):
                        base = r * D
                        na = list(acc)
                        for g in range(8):
                            v = buf[pl.ds(base + 16 * g, 16)]
                            v2 = v * v
                            v4 = v2 * v2
                            na[0 * 8 + g] = na[0 * 8 + g] + v
                            na[1 * 8 + g] = na[1 * 8 + g] + v2
                            na[2 * 8 + g] = na[2 * 8 + g] + v2 * v
                            na[3 * 8 + g] = na[3 * 8 + g] + v4
                            na[4 * 8 + g] = na[4 * 8 + g] + v4 * v
                        return tuple(na)

                    acc = lax.fori_loop(lo, hi, row_body, (zf,) * (NPOW * 8))

                    @pl.when(hi > lo)
                    def _():
                        for j in range(NPOW):
                            for g in range(8):
                                idx = k * SEG_STRIDE + j * D + 16 * g
                                plsc.addupdate(
                                    stage.at[pl.ds(idx, 16)], acc[j * 8 + g])

                    return c

                lax.fori_loop(0, SEG_PER_W, seg_body, 0)

                @pl.when(blk + NBUF < nbw)
                def _():
                    start(blk + NBUF, b)

        return carry

    lax.fori_loop(0, (nbw + NBUF - 1) // NBUF, pair_body, 0)

    pltpu.sync_copy(
        stage, out_hbm.at[pl.ds(wid * SEG_PER_W * SEG_STRIDE, SEG_PER_W * SEG_STRIDE)]
    )


def _sc_moment_sums(x1f, x2f, offsp):
    n1 = x1f.shape[0] // D
    n2 = x2f.shape[0] // D
    mesh = plsc.VectorSubcoreMesh(core_axis_name="c", subcore_axis_name="s")

    @functools.partial(
        pl.kernel,
        out_type=(
            jax.ShapeDtypeStruct((NSEG * SEG_STRIDE,), jnp.float32),
            jax.ShapeDtypeStruct((NSEG * SEG_STRIDE,), jnp.float32),
        ),
        mesh=mesh,
        scratch_types=(
            [pltpu.VMEM((B * D,), jnp.float32)] * NBUF
            + [
                pltpu.VMEM((SEG_PER_W * SEG_STRIDE,), jnp.float32),
                pltpu.VMEM((2 * OFFS_PAD,), jnp.int32),
                pltpu.SMEM((32,), jnp.int32),
            ]
            + [pltpu.SemaphoreType.DMA] * NBUF
        ),
    )
    def sc_kernel(x1_hbm, x2_hbm, o_hbm, s1_hbm, s2_hbm, *rest):
        bufs = rest[:NBUF]
        stage, o_v, offs_s = rest[NBUF:NBUF + 3]
        sems = rest[NBUF + 3:]
        wid = lax.axis_index("s") * NC + lax.axis_index("c")
        pltpu.sync_copy(o_hbm, o_v)
        _accumulate_task(x1_hbm, o_v.at[pl.ds(0, OFFS_PAD)], s1_hbm,
                         bufs, sems, stage, offs_s, n1, wid)
        _accumulate_task(x2_hbm, o_v.at[pl.ds(OFFS_PAD, OFFS_PAD)], s2_hbm,
                         bufs, sems, stage, offs_s, n2, wid)

    return sc_kernel(x1f, x2f, offsp)


def _tc_finish(s1, s2, lo1, hi1, lo2, hi2):
    def body(s1_ref, s2_ref, lo1_ref, hi1_ref, lo2_ref, hi2_ref, out_ref):
        def central(s_ref, lo_ref, hi_ref):
            n = jnp.maximum((hi_ref[...] - lo_ref[...]).astype(jnp.float32), 1.0)
            inv = 1.0 / n
            M1 = s_ref[:, 0 * D:1 * D] * inv
            M2 = s_ref[:, 1 * D:2 * D] * inv
            M3 = s_ref[:, 2 * D:3 * D] * inv
            M4 = s_ref[:, 3 * D:4 * D] * inv
            M5 = s_ref[:, 4 * D:5 * D] * inv
            m = M1
            m2 = m * m
            m3 = m2 * m
            c2 = M2 - m2
            c3 = M3 - 3.0 * m * M2 + 2.0 * m3
            c4 = M4 - 4.0 * m * M3 + 6.0 * m2 * M2 - 3.0 * m2 * m2
            c5 = M5 - 5.0 * m * M4 + 10.0 * m2 * M3 - 10.0 * m3 * M2 + 4.0 * m3 * m2
            return (m, c2, c3, c4, c5)

        A = central(s1_ref, lo1_ref, hi1_ref)
        Bm = central(s2_ref, lo2_ref, hi2_ref)
        tot = jnp.zeros((NSEG, 1), jnp.float32)
        for a, b in zip(A, Bm):
            diff = a - b
            tot = tot + jnp.sqrt(jnp.sum(diff * diff, axis=1, keepdims=True))
        out_ref[...] = (jnp.sum(tot) / NSEG) * jnp.ones((1, 1), jnp.float32)

    return pl.pallas_call(
        body,
        out_shape=jax.ShapeDtypeStruct((1, 1), jnp.float32),
    )(s1, s2, lo1, hi1, lo2, hi2)


def kernel(x1, x2, og_batch, coarse_batch, n_moments):
    ids1 = og_batch.astype(jnp.int32)
    ids2 = coarse_batch.astype(jnp.int32)

    e = _sc_run_ends(_offsets(ids1, CH1), _offsets(ids2, CH2))
    ends = jnp.max(e.reshape(2, NW, ETAB), axis=1)[:, :512]
    offs = jnp.concatenate(
        [jnp.zeros((2, 1), jnp.int32), lax.cummax(ends, axis=1)], axis=1)
    offsp = jnp.zeros((2, OFFS_PAD), jnp.int32).at[:, :513].set(offs)

    s1, s2 = _sc_moment_sums(x1.reshape(-1), x2.reshape(-1),
                             offsp.reshape(-1))

    out = _tc_finish(
        s1.reshape(NSEG, SEG_STRIDE),
        s2.reshape(NSEG, SEG_STRIDE),
        offs[0, :512].reshape(NSEG, 1),
        offs[0, 1:].reshape(NSEG, 1),
        offs[1, :512].reshape(NSEG, 1),
        offs[1, 1:].reshape(NSEG, 1),
    )
    return out[0, 0]


# speedup vs baseline: 2.2663x; 1.0025x over previous
import functools

import jax
import jax.numpy as jnp
from jax import lax
from jax.experimental import pallas as pl
from jax.experimental.pallas import tpu as pltpu
from jax.experimental.pallas import tpu_sc as plsc

NSEG = 512
D = 128
NC = 2
NS = 16
NW = NC * NS
SEG_PER_W = NSEG // NW
NPOW = 5
B = 448
NBUF = 2
SEG_STRIDE = NPOW * D
OFFS_PAD = 544
CH1 = 10000
CH2 = 5008
ETAB = 544


def _sc_run_ends(ids1p, ids2p):
    mesh = plsc.VectorSubcoreMesh(core_axis_name="c", subcore_axis_name="s")

    @functools.partial(
        pl.kernel,
        out_type=jax.ShapeDtypeStruct((2 * NW * ETAB,), jnp.int32),
        mesh=mesh,
        compiler_params=pltpu.CompilerParams(needs_layout_passes=False),
        scratch_types=[
            pltpu.VMEM((CH1 + 16,), jnp.int32),
            pltpu.VMEM((ETAB,), jnp.int32),
        ],
    )
    def k(i1_hbm, i2_hbm, e_hbm, idsv, ends):
        wid = lax.axis_index("s") * NC + lax.axis_index("c")
        iota = lax.iota(jnp.int32, 16)
        zero = jnp.zeros((16,), jnp.int32)
        for t, (src, ch) in enumerate(((i1_hbm, CH1), (i2_hbm, CH2))):
            pltpu.sync_copy(src.at[pl.ds(wid * ch, ch + 16)],
                            idsv.at[pl.ds(0, ch + 16)])
            for i in range(ETAB // 16):
                ends[pl.ds(16 * i, 16)] = zero
            base = wid * ch + 1

            def grp(i, carry, base=base):
                g = idsv[pl.ds(16 * i, 16)]
                gn = idsv[pl.ds(16 * i + 1, 16)]
                pos = iota + (16 * i + base)
                plsc.store_scatter(ends, [g], pos, mask=g != gn)
                return carry

            lax.fori_loop(0, ch // 16, grp, 0)
            pltpu.sync_copy(
                ends, e_hbm.at[pl.ds((t * NW + wid) * ETAB, ETAB)])

    return k(ids1p, ids2p)


def _offsets(ids, chunk):
    n = ids.shape[0]
    pad = NW * chunk + 16 - n
    return jnp.concatenate([ids, jnp.full((pad,), NSEG, jnp.int32)])


def _accumulate_task(x_hbm, offs_v, out_hbm, bufs, sems,
                     stage, offs_s, n_rows, wid):
    g0 = offs_v[pl.ds(wid * 16, 16)]
    g1 = offs_v[pl.ds(wid * 16 + 8, 16)]
    for k in range(SEG_PER_W):
        offs_s[k] = g0[k]
    offs_s[SEG_PER_W] = g1[8]
    rs = g0[0]
    re = g1[8]
    nbw = (re - rs + B - 1) // B

    zf = jnp.zeros((16,), jnp.float32)

    def zero_body(i, c):
        stage[pl.ds(16 * i, 16)] = zf
        return c

    lax.fori_loop(0, SEG_PER_W * SEG_STRIDE // 16, zero_body, 0)

    def start(blk, b):
        w0 = rs + blk * B
        c0 = jnp.minimum(w0, n_rows - B)
        pltpu.async_copy(x_hbm.at[pl.ds(c0 * D, B * D)], bufs[b], sems[b])

    def wait(b):
        pltpu.make_async_copy(x_hbm.at[pl.ds(0, B * D)], bufs[b], sems[b]).wait()

    for b0 in range(NBUF):
        @pl.when(nbw > b0)
        def _(b0=b0):
            start(b0, b0)

    def pair_body(i, carry):
        for b in range(NBUF):
            blk = NBUF * i + b

            @pl.when(blk < nbw)
            def _(blk=blk, b=b):
                buf = bufs[b]
                wait(b)
                w0 = rs + blk * B
                c0 = jnp.minimum(w0, n_rows - B)
                hi_w = w0 + B

                def seg_body(k, c):
                    a = jnp.maximum(offs_s[k], w0)
                    bb = jnp.minimum(offs_s[k + 1], hi_w)
                    lo = a - c0
                    hi = bb - c0

                    def row_body(r, acc):
                        base = r * D
                        na = list(acc)
                        for g in range(8):
                            v = buf[pl.ds(base + 16 * g, 16)]
                            v2 = v * v
                            v4 = v2 * v2
                            na[0 * 8 + g] = na[0 * 8 + g] + v
                            na[1 * 8 + g] = na[1 * 8 + g] + v2
                            na[2 * 8 + g] = na[2 * 8 + g] + v2 * v
                            na[3 * 8 + g] = na[3 * 8 + g] + v4
                            na[4 * 8 + g] = na[4 * 8 + g] + v4 * v
                        return tuple(na)

                    acc = lax.fori_loop(lo, hi, row_body, (zf,) * (NPOW * 8))

                    @pl.when(hi > lo)
                    def _():
                        for j in range(NPOW):
                            for g in range(8):
                                idx = k * SEG_STRIDE + j * D + 16 * g
                                plsc.addupdate(
                                    stage.at[pl.ds(idx, 16)], acc[j * 8 + g])

                    return c

                lax.fori_loop(0, SEG_PER_W, seg_body, 0)

                @pl.when(blk + NBUF < nbw)
                def _():
                    start(blk + NBUF, b)

        return carry

    lax.fori_loop(0, (nbw + NBUF - 1) // NBUF, pair_body, 0)

    pltpu.sync_copy(
        stage, out_hbm.at[pl.ds(wid * SEG_PER_W * SEG_STRIDE, SEG_PER_W * SEG_STRIDE)]
    )


def _sc_moment_sums(x1f, x2f, offsp):
    n1 = x1f.shape[0] // D
    n2 = x2f.shape[0] // D
    mesh = plsc.VectorSubcoreMesh(core_axis_name="c", subcore_axis_name="s")

    @functools.partial(
        pl.kernel,
        out_type=(
            jax.ShapeDtypeStruct((NSEG * SEG_STRIDE,), jnp.float32),
            jax.ShapeDtypeStruct((NSEG * SEG_STRIDE,), jnp.float32),
        ),
        mesh=mesh,
        scratch_types=(
            [pltpu.VMEM((B * D,), jnp.float32)] * NBUF
            + [
                pltpu.VMEM((SEG_PER_W * SEG_STRIDE,), jnp.float32),
                pltpu.VMEM((2 * OFFS_PAD,), jnp.int32),
                pltpu.SMEM((32,), jnp.int32),
            ]
            + [pltpu.SemaphoreType.DMA] * NBUF
        ),
    )
    def sc_kernel(x1_hbm, x2_hbm, o_hbm, s1_hbm, s2_hbm, *rest):
        bufs = rest[:NBUF]
        stage, o_v, offs_s = rest[NBUF:NBUF + 3]
        sems = rest[NBUF + 3:]
        wid = lax.axis_index("s") * NC + lax.axis_index("c")
        pltpu.sync_copy(o_hbm, o_v)
        _accumulate_task(x1_hbm, o_v.at[pl.ds(0, OFFS_PAD)], s1_hbm,
                         bufs, sems, stage, offs_s, n1, wid)
        _accumulate_task(x2_hbm, o_v.at[pl.ds(OFFS_PAD, OFFS_PAD)], s2_hbm,
                         bufs, sems, stage, offs_s, n2, wid)

    return sc_kernel(x1f, x2f, offsp)


def _tc_finish(s1, s2, lo1, hi1, lo2, hi2):
    def body(s1_ref, s2_ref, lo1_ref, hi1_ref, lo2_ref, hi2_ref, out_ref):
        def central(s_ref, lo_ref, hi_ref):
            n = jnp.maximum((hi_ref[...] - lo_ref[...]).astype(jnp.float32), 1.0)
            inv = 1.0 / n
            M1 = s_ref[:, 0 * D:1 * D] * inv
            M2 = s_ref[:, 1 * D:2 * D] * inv
            M3 = s_ref[:, 2 * D:3 * D] * inv
            M4 = s_ref[:, 3 * D:4 * D] * inv
            M5 = s_ref[:, 4 * D:5 * D] * inv
            m = M1
            m2 = m * m
            m3 = m2 * m
            c2 = M2 - m2
            c3 = M3 - 3.0 * m * M2 + 2.0 * m3
            c4 = M4 - 4.0 * m * M3 + 6.0 * m2 * M2 - 3.0 * m2 * m2
            c5 = M5 - 5.0 * m * M4 + 10.0 * m2 * M3 - 10.0 * m3 * M2 + 4.0 * m3 * m2
            return (m, c2, c3, c4, c5)

        A = central(s1_ref, lo1_ref, hi1_ref)
        Bm = central(s2_ref, lo2_ref, hi2_ref)
        tot = jnp.zeros((NSEG, 1), jnp.float32)
        for a, b in zip(A, Bm):
            diff = a - b
            tot = tot + jnp.sqrt(jnp.sum(diff * diff, axis=1, keepdims=True))
        out_ref[...] = (jnp.sum(tot) / NSEG) * jnp.ones((1, 1), jnp.float32)

    return pl.pallas_call(
        body,
        out_shape=jax.ShapeDtypeStruct((1, 1), jnp.float32),
    )(s1, s2, lo1, hi1, lo2, hi2)


def kernel(x1, x2, og_batch, coarse_batch, n_moments):
    ids1 = og_batch.astype(jnp.int32)
    ids2 = coarse_batch.astype(jnp.int32)

    e = _sc_run_ends(_offsets(ids1, CH1), _offsets(ids2, CH2))
    ends = jnp.max(e.reshape(2, NW, ETAB), axis=1)[:, :512]
    offs = jnp.concatenate(
        [jnp.zeros((2, 1), jnp.int32), lax.cummax(ends, axis=1)], axis=1)
    offsp = jnp.zeros((2, OFFS_PAD), jnp.int32).at[:, :513].set(offs)

    s1, s2 = _sc_moment_sums(x1.reshape(-1), x2.reshape(-1),
                             offsp.reshape(-1))

    out = _tc_finish(
        s1.reshape(NSEG, SEG_STRIDE),
        s2.reshape(NSEG, SEG_STRIDE),
        offs[0, :512].reshape(NSEG, 1),
        offs[0, 1:].reshape(NSEG, 1),
        offs[1, :512].reshape(NSEG, 1),
        offs[1, 1:].reshape(NSEG, 1),
    )
    return out[0, 0]


# speedup vs baseline: 2.3030x; 1.0162x over previous
import functools

import jax
import jax.numpy as jnp
from jax import lax
from jax.experimental import pallas as pl
from jax.experimental.pallas import tpu as pltpu
from jax.experimental.pallas import tpu_sc as plsc

NSEG = 512
D = 128
NC = 2
NS = 16
NW = NC * NS
SEG_PER_W = NSEG // NW
NPOW = 5
B = 448
NBUF = 2
SEG_STRIDE = NPOW * D
OFFS_PAD = 544
CH1 = 10000
CH2 = 5008
ETAB = 544


def _sc_run_ends(ids1p, ids2p):
    mesh = plsc.VectorSubcoreMesh(core_axis_name="c", subcore_axis_name="s")

    @functools.partial(
        pl.kernel,
        out_type=jax.ShapeDtypeStruct((2 * NW * ETAB,), jnp.int32),
        mesh=mesh,
        compiler_params=pltpu.CompilerParams(needs_layout_passes=False),
        scratch_types=[
            pltpu.VMEM((CH1 + 16,), jnp.int32),
            pltpu.VMEM((ETAB,), jnp.int32),
        ],
    )
    def k(i1_hbm, i2_hbm, e_hbm, idsv, ends):
        wid = lax.axis_index("s") * NC + lax.axis_index("c")
        iota = lax.iota(jnp.int32, 16)
        zero = jnp.zeros((16,), jnp.int32)
        for t, (src, ch) in enumerate(((i1_hbm, CH1), (i2_hbm, CH2))):
            pltpu.sync_copy(src.at[pl.ds(wid * ch, ch + 16)],
                            idsv.at[pl.ds(0, ch + 16)])
            for i in range(ETAB // 16):
                ends[pl.ds(16 * i, 16)] = zero
            base = wid * ch + 1

            def grp(i, carry, base=base):
                g = idsv[pl.ds(16 * i, 16)]
                gn = idsv[pl.ds(16 * i + 1, 16)]
                pos = iota + (16 * i + base)
                plsc.store_scatter(ends, [g], pos, mask=g != gn)
                return carry

            lax.fori_loop(0, ch // 16, grp, 0)
            pltpu.sync_copy(
                ends, e_hbm.at[pl.ds((t * NW + wid) * ETAB, ETAB)])

    return k(ids1p, ids2p)


def _offsets(ids, chunk):
    n = ids.shape[0]
    pad = NW * chunk + 16 - n
    return jnp.concatenate([ids, jnp.full((pad,), NSEG, jnp.int32)])


def _accumulate_task(x_hbm, offs_v, out_hbm, bufs, sems,
                     stage, offs_s, n_rows, wid):
    g0 = offs_v[pl.ds(wid * 16, 16)]
    g1 = offs_v[pl.ds(wid * 16 + 8, 16)]
    for k in range(SEG_PER_W):
        offs_s[k] = g0[k]
    offs_s[SEG_PER_W] = g1[8]
    rs = g0[0]
    re = g1[8]
    nbw = (re - rs + B - 1) // B

    zf = jnp.zeros((16,), jnp.float32)

    def zero_body(i, c):
        stage[pl.ds(16 * i, 16)] = zf
        return c

    lax.fori_loop(0, SEG_PER_W * SEG_STRIDE // 16, zero_body, 0)

    def start(blk, b):
        w0 = rs + blk * B
        c0 = jnp.minimum(w0, n_rows - B)
        pltpu.async_copy(x_hbm.at[pl.ds(c0 * D, B * D)], bufs[b], sems[b])

    def wait(b):
        pltpu.make_async_copy(x_hbm.at[pl.ds(0, B * D)], bufs[b], sems[b]).wait()

    for b0 in range(NBUF):
        @pl.when(nbw > b0)
        def _(b0=b0):
            start(b0, b0)

    def pair_body(i, kcur):
        for b in range(NBUF):
            blk = NBUF * i + b

            def do_block(kcur, blk=blk, b=b):
                buf = bufs[b]
                wait(b)
                w0 = rs + blk * B
                c0 = jnp.minimum(w0, n_rows - B)
                hi_w = w0 + B

                def adv_body(st):
                    k, _ = st
                    return (k + 1, offs_s[k + 2])

                kcur, _ = lax.while_loop(
                    lambda st: jnp.logical_and(st[0] < SEG_PER_W - 1,
                                               st[1] <= w0),
                    adv_body,
                    (kcur, offs_s[kcur + 1]),
                )

                def seg_body(st):
                    k, a0 = st
                    a = jnp.maximum(a0, w0)
                    bb = jnp.minimum(offs_s[k + 1], hi_w)
                    lo = a - c0
                    hi = bb - c0

                    def row_body(r, acc):
                        base = r * D
                        na = list(acc)
                        for g in range(8):
                            v = buf[pl.ds(base + 16 * g, 16)]
                            v2 = v * v
                            v4 = v2 * v2
                            na[0 * 8 + g] = na[0 * 8 + g] + v
                            na[1 * 8 + g] = na[1 * 8 + g] + v2
                            na[2 * 8 + g] = na[2 * 8 + g] + v2 * v
                            na[3 * 8 + g] = na[3 * 8 + g] + v4
                            na[4 * 8 + g] = na[4 * 8 + g] + v4 * v
                        return tuple(na)

                    acc = lax.fori_loop(lo, hi, row_body, (zf,) * (NPOW * 8))

                    @pl.when(hi > lo)
                    def _():
                        for j in range(NPOW):
                            for g in range(8):
                                idx = k * SEG_STRIDE + j * D + 16 * g
                                plsc.addupdate(
                                    stage.at[pl.ds(idx, 16)], acc[j * 8 + g])

                    return (k + 1, offs_s[k + 1])

                k_end, _ = lax.while_loop(
                    lambda st: jnp.logical_and(st[0] < SEG_PER_W,
                                               st[1] < hi_w),
                    seg_body,
                    (kcur, offs_s[kcur]),
                )

                @pl.when(blk + NBUF < nbw)
                def _():
                    start(blk + NBUF, b)

                return kcur

            kcur = lax.cond(blk < nbw, do_block, lambda kcur: kcur, kcur)

        return kcur

    lax.fori_loop(0, (nbw + NBUF - 1) // NBUF, pair_body, 0)

    pltpu.sync_copy(
        stage, out_hbm.at[pl.ds(wid * SEG_PER_W * SEG_STRIDE, SEG_PER_W * SEG_STRIDE)]
    )


def _sc_moment_sums(x1f, x2f, offsp):
    n1 = x1f.shape[0] // D
    n2 = x2f.shape[0] // D
    mesh = plsc.VectorSubcoreMesh(core_axis_name="c", subcore_axis_name="s")

    @functools.partial(
        pl.kernel,
        out_type=(
            jax.ShapeDtypeStruct((NSEG * SEG_STRIDE,), jnp.float32),
            jax.ShapeDtypeStruct((NSEG * SEG_STRIDE,), jnp.float32),
        ),
        mesh=mesh,
        compiler_params=pltpu.CompilerParams(needs_layout_passes=False),
        scratch_types=(
            [pltpu.VMEM((B * D,), jnp.float32)] * NBUF
            + [
                pltpu.VMEM((SEG_PER_W * SEG_STRIDE,), jnp.float32),
                pltpu.VMEM((2 * OFFS_PAD,), jnp.int32),
                pltpu.SMEM((32,), jnp.int32),
            ]
            + [pltpu.SemaphoreType.DMA] * NBUF
        ),
    )
    def sc_kernel(x1_hbm, x2_hbm, o_hbm, s1_hbm, s2_hbm, *rest):
        bufs = rest[:NBUF]
        stage, o_v, offs_s = rest[NBUF:NBUF + 3]
        sems = rest[NBUF + 3:]
        wid = lax.axis_index("s") * NC + lax.axis_index("c")
        pltpu.sync_copy(o_hbm, o_v)
        _accumulate_task(x1_hbm, o_v.at[pl.ds(0, OFFS_PAD)], s1_hbm,
                         bufs, sems, stage, offs_s, n1, wid)
        _accumulate_task(x2_hbm, o_v.at[pl.ds(OFFS_PAD, OFFS_PAD)], s2_hbm,
                         bufs, sems, stage, offs_s, n2, wid)

    return sc_kernel(x1f, x2f, offsp)


def _tc_finish(s1, s2, lo1, hi1, lo2, hi2):
    def body(s1_ref, s2_ref, lo1_ref, hi1_ref, lo2_ref, hi2_ref, out_ref):
        def central(s_ref, lo_ref, hi_ref):
            n = jnp.maximum((hi_ref[...] - lo_ref[...]).astype(jnp.float32), 1.0)
            inv = 1.0 / n
            M1 = s_ref[:, 0 * D:1 * D] * inv
            M2 = s_ref[:, 1 * D:2 * D] * inv
            M3 = s_ref[:, 2 * D:3 * D] * inv
            M4 = s_ref[:, 3 * D:4 * D] * inv
            M5 = s_ref[:, 4 * D:5 * D] * inv
            m = M1
            m2 = m * m
            m3 = m2 * m
            c2 = M2 - m2
            c3 = M3 - 3.0 * m * M2 + 2.0 * m3
            c4 = M4 - 4.0 * m * M3 + 6.0 * m2 * M2 - 3.0 * m2 * m2
            c5 = M5 - 5.0 * m * M4 + 10.0 * m2 * M3 - 10.0 * m3 * M2 + 4.0 * m3 * m2
            return (m, c2, c3, c4, c5)

        A = central(s1_ref, lo1_ref, hi1_ref)
        Bm = central(s2_ref, lo2_ref, hi2_ref)
        tot = jnp.zeros((NSEG, 1), jnp.float32)
        for a, b in zip(A, Bm):
            diff = a - b
            tot = tot + jnp.sqrt(jnp.sum(diff * diff, axis=1, keepdims=True))
        out_ref[...] = (jnp.sum(tot) / NSEG) * jnp.ones((1, 1), jnp.float32)

    return pl.pallas_call(
        body,
        out_shape=jax.ShapeDtypeStruct((1, 1), jnp.float32),
    )(s1, s2, lo1, hi1, lo2, hi2)


def kernel(x1, x2, og_batch, coarse_batch, n_moments):
    ids1 = og_batch.astype(jnp.int32)
    ids2 = coarse_batch.astype(jnp.int32)

    e = _sc_run_ends(_offsets(ids1, CH1), _offsets(ids2, CH2))
    ends = jnp.max(e.reshape(2, NW, ETAB), axis=1)[:, :512]
    offs = jnp.concatenate(
        [jnp.zeros((2, 1), jnp.int32), lax.cummax(ends, axis=1)], axis=1)
    offsp = jnp.zeros((2, OFFS_PAD), jnp.int32).at[:, :513].set(offs)

    s1, s2 = _sc_moment_sums(x1.reshape(-1), x2.reshape(-1),
                             offsp.reshape(-1))

    out = _tc_finish(
        s1.reshape(NSEG, SEG_STRIDE),
        s2.reshape(NSEG, SEG_STRIDE),
        offs[0, :512].reshape(NSEG, 1),
        offs[0, 1:].reshape(NSEG, 1),
        offs[1, :512].reshape(NSEG, 1),
        offs[1, 1:].reshape(NSEG, 1),
    )
    return out[0, 0]


# speedup vs baseline: 2.3034x; 1.0002x over previous
import functools

import jax
import jax.numpy as jnp
from jax import lax
from jax.experimental import pallas as pl
from jax.experimental.pallas import tpu as pltpu
from jax.experimental.pallas import tpu_sc as plsc

NSEG = 512
D = 128
NC = 2
NS = 16
NW = NC * NS
SEG_PER_W = NSEG // NW
NPOW = 5
B = 448
NBUF = 2
SEG_STRIDE = NPOW * D
OFFS_PAD = 544
CH1 = 10000
CH2 = 5008
ETAB = 544


def _sc_run_ends(ids1p, ids2p):
    mesh = plsc.VectorSubcoreMesh(core_axis_name="c", subcore_axis_name="s")

    @functools.partial(
        pl.kernel,
        out_type=jax.ShapeDtypeStruct((2 * NW * ETAB,), jnp.int32),
        mesh=mesh,
        compiler_params=pltpu.CompilerParams(needs_layout_passes=False),
        scratch_types=[
            pltpu.VMEM((CH1 + 16,), jnp.int32),
            pltpu.VMEM((ETAB,), jnp.int32),
        ],
    )
    def k(i1_hbm, i2_hbm, e_hbm, idsv, ends):
        wid = lax.axis_index("s") * NC + lax.axis_index("c")
        iota = lax.iota(jnp.int32, 16)
        zero = jnp.zeros((16,), jnp.int32)
        for t, (src, ch) in enumerate(((i1_hbm, CH1), (i2_hbm, CH2))):
            pltpu.sync_copy(src.at[pl.ds(wid * ch, ch + 16)],
                            idsv.at[pl.ds(0, ch + 16)])
            for i in range(ETAB // 16):
                ends[pl.ds(16 * i, 16)] = zero
            base = wid * ch + 1

            def grp(i, carry, base=base):
                g = idsv[pl.ds(16 * i, 16)]
                gn = idsv[pl.ds(16 * i + 1, 16)]
                pos = iota + (16 * i + base)
                plsc.store_scatter(ends, [g], pos, mask=g != gn)
                return carry

            lax.fori_loop(0, ch // 16, grp, 0)
            pltpu.sync_copy(
                ends, e_hbm.at[pl.ds((t * NW + wid) * ETAB, ETAB)])

    return k(ids1p, ids2p)


def _pad_ids(ids, chunk):
    n = ids.shape[0]
    pad = NW * chunk + 16 - n
    return jnp.concatenate([ids, jnp.full((pad,), NSEG, jnp.int32)])


def _accumulate_task(x_hbm, offs_v, out_hbm, bufs, sems,
                     stage, offs_s, n_rows, wid):
    g0 = offs_v[pl.ds(wid * 16, 16)]
    g1 = offs_v[pl.ds(wid * 16 + 8, 16)]
    for k in range(SEG_PER_W):
        offs_s[k] = g0[k]
    offs_s[SEG_PER_W] = g1[8]
    rs = g0[0]
    re = g1[8]
    nbw = (re - rs + B - 1) // B

    zf = jnp.zeros((16,), jnp.float32)

    def zero_body(i, c):
        stage[pl.ds(16 * i, 16)] = zf
        return c

    lax.fori_loop(0, SEG_PER_W * SEG_STRIDE // 16, zero_body, 0)

    def start(blk, b):
        w0 = rs + blk * B
        c0 = jnp.minimum(w0, n_rows - B)
        pltpu.async_copy(x_hbm.at[pl.ds(c0 * D, B * D)], bufs[b], sems[b])

    def wait(b):
        pltpu.make_async_copy(x_hbm.at[pl.ds(0, B * D)], bufs[b], sems[b]).wait()

    for b0 in range(NBUF):
        @pl.when(nbw > b0)
        def _(b0=b0):
            start(b0, b0)

    def pair_body(i, kcur):
        for b in range(NBUF):
            blk = NBUF * i + b

            def do_block(kcur, blk=blk, b=b):
                buf = bufs[b]
                wait(b)
                w0 = rs + blk * B
                c0 = jnp.minimum(w0, n_rows - B)
                hi_w = w0 + B

                def adv_body(st):
                    k, _ = st
                    return (k + 1, offs_s[k + 2])

                kcur, _ = lax.while_loop(
                    lambda st: jnp.logical_and(st[0] < SEG_PER_W - 1,
                                               st[1] <= w0),
                    adv_body,
                    (kcur, offs_s[kcur + 1]),
                )

                def seg_body(st):
                    k, a0 = st
                    a = jnp.maximum(a0, w0)
                    bb = jnp.minimum(offs_s[k + 1], hi_w)
                    lo = a - c0
                    hi = bb - c0

                    def row_body(r, acc):
                        base = r * D
                        na = list(acc)
                        for g in range(8):
                            v = buf[pl.ds(base + 16 * g, 16)]
                            v2 = v * v
                            v4 = v2 * v2
                            na[0 * 8 + g] = na[0 * 8 + g] + v
                            na[1 * 8 + g] = na[1 * 8 + g] + v2
                            na[2 * 8 + g] = na[2 * 8 + g] + v2 * v
                            na[3 * 8 + g] = na[3 * 8 + g] + v4
                            na[4 * 8 + g] = na[4 * 8 + g] + v4 * v
                        return tuple(na)

                    acc = lax.fori_loop(lo, hi, row_body, (zf,) * (NPOW * 8))

                    @pl.when(hi > lo)
                    def _():
                        for j in range(NPOW):
                            for g in range(8):
                                idx = k * SEG_STRIDE + j * D + 16 * g
                                plsc.addupdate(
                                    stage.at[pl.ds(idx, 16)], acc[j * 8 + g])

                    return (k + 1, offs_s[k + 1])

                k_end, _ = lax.while_loop(
                    lambda st: jnp.logical_and(st[0] < SEG_PER_W,
                                               st[1] < hi_w),
                    seg_body,
                    (kcur, offs_s[kcur]),
                )

                @pl.when(blk + NBUF < nbw)
                def _():
                    start(blk + NBUF, b)

                return kcur

            kcur = lax.cond(blk < nbw, do_block, lambda kcur: kcur, kcur)

        return kcur

    lax.fori_loop(0, (nbw + NBUF - 1) // NBUF, pair_body, 0)

    pltpu.sync_copy(
        stage, out_hbm.at[pl.ds(wid * SEG_PER_W * SEG_STRIDE, SEG_PER_W * SEG_STRIDE)]
    )


def _sc_moment_sums(x1f, x2f, offsp):
    n1 = x1f.shape[0] // D
    n2 = x2f.shape[0] // D
    mesh = plsc.VectorSubcoreMesh(core_axis_name="c", subcore_axis_name="s")

    @functools.partial(
        pl.kernel,
        out_type=(
            jax.ShapeDtypeStruct((NSEG * SEG_STRIDE,), jnp.float32),
            jax.ShapeDtypeStruct((NSEG * SEG_STRIDE,), jnp.float32),
        ),
        mesh=mesh,
        compiler_params=pltpu.CompilerParams(needs_layout_passes=False),
        scratch_types=(
            [pltpu.VMEM((B * D,), jnp.float32)] * NBUF
            + [
                pltpu.VMEM((SEG_PER_W * SEG_STRIDE,), jnp.float32),
                pltpu.VMEM((2 * OFFS_PAD,), jnp.int32),
                pltpu.SMEM((32,), jnp.int32),
            ]
            + [pltpu.SemaphoreType.DMA] * NBUF
        ),
    )
    def sc_kernel(x1_hbm, x2_hbm, o_hbm, s1_hbm, s2_hbm, *rest):
        bufs = rest[:NBUF]
        stage, o_v, offs_s = rest[NBUF:NBUF + 3]
        sems = rest[NBUF + 3:]
        wid = lax.axis_index("s") * NC + lax.axis_index("c")
        pltpu.sync_copy(o_hbm, o_v)
        _accumulate_task(x1_hbm, o_v.at[pl.ds(0, OFFS_PAD)], s1_hbm,
                         bufs, sems, stage, offs_s, n1, wid)
        _accumulate_task(x2_hbm, o_v.at[pl.ds(OFFS_PAD, OFFS_PAD)], s2_hbm,
                         bufs, sems, stage, offs_s, n2, wid)

    return sc_kernel(x1f, x2f, offsp)


def _tc_finish(s1, s2, lo1, hi1, lo2, hi2):
    def body(s1_ref, s2_ref, lo1_ref, hi1_ref, lo2_ref, hi2_ref, out_ref):
        def central(s_ref, lo_ref, hi_ref):
            n = jnp.maximum((hi_ref[...] - lo_ref[...]).astype(jnp.float32), 1.0)
            inv = 1.0 / n
            M1 = s_ref[:, 0 * D:1 * D] * inv
            M2 = s_ref[:, 1 * D:2 * D] * inv
            M3 = s_ref[:, 2 * D:3 * D] * inv
            M4 = s_ref[:, 3 * D:4 * D] * inv
            M5 = s_ref[:, 4 * D:5 * D] * inv
            m = M1
            m2 = m * m
            m3 = m2 * m
            c2 = M2 - m2
            c3 = M3 - 3.0 * m * M2 + 2.0 * m3
            c4 = M4 - 4.0 * m * M3 + 6.0 * m2 * M2 - 3.0 * m2 * m2
            c5 = M5 - 5.0 * m * M4 + 10.0 * m2 * M3 - 10.0 * m3 * M2 + 4.0 * m3 * m2
            return (m, c2, c3, c4, c5)

        A = central(s1_ref, lo1_ref, hi1_ref)
        Bm = central(s2_ref, lo2_ref, hi2_ref)
        tot = jnp.zeros((NSEG, 1), jnp.float32)
        for a, b in zip(A, Bm):
            diff = a - b
            tot = tot + jnp.sqrt(jnp.sum(diff * diff, axis=1, keepdims=True))
        out_ref[...] = (jnp.sum(tot) / NSEG) * jnp.ones((1, 1), jnp.float32)

    return pl.pallas_call(
        body,
        out_shape=jax.ShapeDtypeStruct((1, 1), jnp.float32),
    )(s1, s2, lo1, hi1, lo2, hi2)


def kernel(x1, x2, og_batch, coarse_batch, n_moments):
    ids1 = og_batch.astype(jnp.int32)
    ids2 = coarse_batch.astype(jnp.int32)

    e = _sc_run_ends(_pad_ids(ids1, CH1), _pad_ids(ids2, CH2))
    ends = jnp.max(e.reshape(2, NW, ETAB), axis=1)[:, :512]
    offs = jnp.concatenate(
        [jnp.zeros((2, 1), jnp.int32), lax.cummax(ends, axis=1)], axis=1)
    offsp = jnp.zeros((2, OFFS_PAD), jnp.int32).at[:, :513].set(offs)

    s1, s2 = _sc_moment_sums(x1.reshape(-1), x2.reshape(-1),
                             offsp.reshape(-1))

    out = _tc_finish(
        s1.reshape(NSEG, SEG_STRIDE),
        s2.reshape(NSEG, SEG_STRIDE),
        offs[0, :512].reshape(NSEG, 1),
        offs[0, 1:].reshape(NSEG, 1),
        offs[1, :512].reshape(NSEG, 1),
        offs[1, 1:].reshape(NSEG, 1),
    )
    return out[0, 0]
